# baseline (speedup 1.0000x reference)
import functools

import jax
import jax.numpy as jnp
from jax import lax
from jax.experimental import pallas as pl
from jax.experimental.pallas import tpu as pltpu
from jax.experimental.pallas import tpu_sc as plsc

N = 10000
E = 160000
G = 64
NC = 2
NS = 16
NW = NC * NS
CHUNK = 125
NCHUNKS = E // CHUNK
KPW = NCHUNKS // NW
NPAD = 10240
RPT = NPAD // NS

_HIGH = jax.lax.Precision.HIGHEST


def _sc_mesh():
    return plsc.VectorSubcoreMesh(core_axis_name="c", subcore_axis_name="s")


def _deg_call(dst2d, ones_rows, zrows):
    def body(dst_hbm, ones_hbm, z_hbm, out_hbm, didx, ones_v, sem, acc):
        c = lax.axis_index("c")
        s = lax.axis_index("s")
        w = s * NC + c
        pltpu.sync_copy(z_hbm, acc.at[pl.ds(s * RPT, RPT)])
        pltpu.sync_copy(dst_hbm.at[pl.ds(w * KPW, KPW)], didx)
        pltpu.sync_copy(ones_hbm, ones_v)
        plsc.subcore_barrier()

        def fire(k, carry):
            pltpu.async_copy(ones_v, acc.at[didx.at[k]], sem, add=True)
            return carry

        def drain(k, carry):
            pltpu.make_async_copy(ones_v, acc.at[didx.at[k]], sem).wait()
            return carry

        lax.fori_loop(0, KPW, fire, 0)
        lax.fori_loop(0, KPW, drain, 0)
        plsc.subcore_barrier()
        pltpu.sync_copy(acc.at[pl.ds(s * RPT, RPT)],
                        out_hbm.at[c, pl.ds(s * RPT, RPT)])

    call = pl.kernel(
        body,
        out_type=jax.ShapeDtypeStruct((NC, NPAD, 16), jnp.float32),
        mesh=_sc_mesh(),
        compiler_params=pltpu.CompilerParams(use_tc_tiling_on_sc=False),
        scratch_types=[
            pltpu.VMEM((KPW, CHUNK), jnp.int32),
            pltpu.VMEM((CHUNK, 16), jnp.float32),
            pltpu.SemaphoreType.DMA,
            pltpu.VMEM_SHARED((NPAD, 16), jnp.float32),
        ],
    )
    return call(dst2d, ones_rows, zrows)


def _agg_call(t, src2d, dst2d, zrows, d):
    _RING = 2 if d == 128 else 4

    def body(t_hbm, src_hbm, dst_hbm, z_hbm, out_hbm, *rest):
        sidx, didx = rest[0], rest[1]
        bufs = rest[2:2 + _RING]
        gs, ss, acc = rest[2 + _RING], rest[3 + _RING], rest[4 + _RING]
        c = lax.axis_index("c")
        s = lax.axis_index("s")
        w = s * NC + c
        pltpu.sync_copy(z_hbm, acc.at[pl.ds(s * RPT, RPT)])
        pltpu.sync_copy(src_hbm.at[pl.ds(w * KPW, KPW)], sidx)
        pltpu.sync_copy(dst_hbm.at[pl.ds(w * KPW, KPW)], didx)
        plsc.subcore_barrier()

        for p in range(_RING):
            pltpu.async_copy(t_hbm.at[sidx.at[p]], bufs[p], gs.at[p])

        def block(j, carry):
            for p in range(_RING):
                k = _RING * j + p
                pltpu.make_async_copy(t_hbm.at[sidx.at[k]], bufs[p],
                                      gs.at[p]).wait()
                pltpu.async_copy(bufs[p], acc.at[didx.at[k]], ss.at[p],
                                 add=True)

                @pl.when(k + _RING < KPW)
                def _():
                    pltpu.make_async_copy(bufs[p], acc.at[didx.at[k]],
                                          ss.at[p]).wait()
                    pltpu.async_copy(t_hbm.at[sidx.at[k + _RING]], bufs[p],
                                     gs.at[p])
            return carry

        lax.fori_loop(0, KPW // _RING, block, 0)
        for p in range(_RING):
            k = KPW - _RING + p
            pltpu.make_async_copy(bufs[p], acc.at[didx.at[k]], ss.at[p]).wait()
        plsc.subcore_barrier()
        pltpu.sync_copy(acc.at[pl.ds(s * RPT, RPT)],
                        out_hbm.at[c, pl.ds(s * RPT, RPT)])

    call = pl.kernel(
        body,
        out_type=jax.ShapeDtypeStruct((NC, NPAD, d), jnp.float32),
        mesh=_sc_mesh(),
        compiler_params=pltpu.CompilerParams(use_tc_tiling_on_sc=False),
        scratch_types=(
            [pltpu.VMEM((KPW, CHUNK), jnp.int32)] * 2
            + [pltpu.VMEM((CHUNK, d), jnp.float32)] * _RING
            + [pltpu.SemaphoreType.DMA((_RING,)),
               pltpu.SemaphoreType.DMA((_RING,)),
               pltpu.VMEM_SHARED((NPAD, d), jnp.float32)]
        ),
    )
    return call(t, src2d, dst2d, zrows)


_BLK = 2000
_GRID = N // _BLK


def _bf16_dot(a, b):
    return jnp.dot(a.astype(jnp.bfloat16), b.astype(jnp.bfloat16),
                   preferred_element_type=jnp.float32)


def _g1_body(x_ref, w1_ref, g1_ref):
    g1_ref[...] = _bf16_dot(x_ref[...], w1_ref[...])


def _g1_call(xpad, W1p):
    return pl.pallas_call(
        _g1_body,
        grid=(_GRID,),
        in_specs=[
            pl.BlockSpec((_BLK, 16), lambda i: (i, 0)),
            pl.BlockSpec((16, 128), lambda i: (0, 0)),
        ],
        out_specs=pl.BlockSpec((_BLK, 128), lambda i: (i, 0)),
        out_shape=jax.ShapeDtypeStruct((N, 128), jnp.float32),
    )(xpad, W1p)


def _prep_body(degp_ref, g1_ref, dinv_ref, t1_ref):
    deg = degp_ref[0, :, 0:1] + degp_ref[1, :, 0:1] + 1.0
    dinv = lax.rsqrt(deg)
    dinv_ref[...] = dinv
    t1_ref[...] = dinv * g1_ref[...]


def _prep_call(degp, g1):
    return pl.pallas_call(
        _prep_body,
        grid=(_GRID,),
        in_specs=[
            pl.BlockSpec((NC, _BLK, 16), lambda i: (0, i, 0)),
            pl.BlockSpec((_BLK, 128), lambda i: (i, 0)),
        ],
        out_specs=[
            pl.BlockSpec((_BLK, 1), lambda i: (i, 0)),
            pl.BlockSpec((_BLK, 128), lambda i: (i, 0)),
        ],
        out_shape=[
            jax.ShapeDtypeStruct((N, 1), jnp.float32),
            jax.ShapeDtypeStruct((N, 128), jnp.float32),
        ],
    )(degp, g1)


def _mid1_body(a1_ref, t1_ref, dinv_ref, b1_ref, w2_ref, t2_ref):
    agg = a1_ref[0] + a1_ref[1] + t1_ref[...]
    dinv = dinv_ref[...]
    h1 = jnp.maximum(dinv * agg + b1_ref[...], 0.0)
    t2_ref[...] = dinv * _bf16_dot(h1, w2_ref[...])


def _mid1_call(a1, t1, dinv, b1, W2):
    return pl.pallas_call(
        _mid1_body,
        grid=(_GRID,),
        in_specs=[
            pl.BlockSpec((NC, _BLK, 128), lambda i: (0, i, 0)),
            pl.BlockSpec((_BLK, 128), lambda i: (i, 0)),
            pl.BlockSpec((_BLK, 1), lambda i: (i, 0)),
            pl.BlockSpec((1, 128), lambda i: (0, 0)),
            pl.BlockSpec((128, 128), lambda i: (0, 0)),
        ],
        out_specs=pl.BlockSpec((_BLK, 128), lambda i: (i, 0)),
        out_shape=jax.ShapeDtypeStruct((N, 128), jnp.float32),
    )(a1, t1, dinv, b1, W2)


def _mid2_body(a2_ref, t2_ref, dinv_ref, b2_ref, w3_ref, t3_ref):
    agg = a2_ref[0] + a2_ref[1] + t2_ref[...]
    dinv = dinv_ref[...]
    h2 = jnp.maximum(dinv * agg + b2_ref[...], 0.0)
    t3_ref[...] = dinv * _bf16_dot(h2, w3_ref[...])


def _mid2_call(a2, t2, dinv, b2, W3):
    return pl.pallas_call(
        _mid2_body,
        grid=(_GRID,),
        in_specs=[
            pl.BlockSpec((NC, _BLK, 128), lambda i: (0, i, 0)),
            pl.BlockSpec((_BLK, 128), lambda i: (i, 0)),
            pl.BlockSpec((_BLK, 1), lambda i: (i, 0)),
            pl.BlockSpec((1, 128), lambda i: (0, 0)),
            pl.BlockSpec((128, 64), lambda i: (0, 0)),
        ],
        out_specs=pl.BlockSpec((_BLK, 64), lambda i: (i, 0)),
        out_shape=jax.ShapeDtypeStruct((N, 64), jnp.float32),
    )(a2, t2, dinv, b2, W3)


def _pool_body(a3_ref, t3_ref, dinv_ref, b3_ref, wl_ref, bl_ref, batch_ref,
               ones_ref, sums_ref, cnts_ref, out_ref):
    i = pl.program_id(0)
    agg = a3_ref[0] + a3_ref[1] + t3_ref[...]
    h3 = jnp.maximum(dinv_ref[...] * agg + b3_ref[...], 0.0)
    gid = lax.broadcasted_iota(jnp.int32, (_BLK, G), 1)
    onehot = (batch_ref[...] == gid).astype(jnp.float32)
    cdims = (((0,), (0,)), ((), ()))
    sm = lax.dot_general(onehot, h3, cdims, precision=_HIGH)
    ct = lax.dot_general(onehot, ones_ref[...], cdims, precision=_HIGH)

    @pl.when(i == 0)
    def _():
        sums_ref[...] = sm
        cnts_ref[...] = ct

    @pl.when(i > 0)
    def _():
        sums_ref[...] += sm
        cnts_ref[...] += ct

    @pl.when(i == pl.num_programs(0) - 1)
    def _():
        pooled = sums_ref[...] / jnp.maximum(cnts_ref[...], 1.0)
        out_ref[...] = _bf16_dot(pooled, wl_ref[...]) + bl_ref[...]


def _pool_call(a3, t3, dinv, b3, Wl, bl, batch2d, ones_col):
    zero = lambda i: (0, 0)
    return pl.pallas_call(
        _pool_body,
        grid=(_GRID,),
        in_specs=[
            pl.BlockSpec((NC, _BLK, 64), lambda i: (0, i, 0)),
            pl.BlockSpec((_BLK, 64), lambda i: (i, 0)),
            pl.BlockSpec((_BLK, 1), lambda i: (i, 0)),
            pl.BlockSpec((1, 64), zero),
            pl.BlockSpec((64, 1), zero),
            pl.BlockSpec((1, 1), zero),
            pl.BlockSpec((_BLK, 1), lambda i: (i, 0)),
            pl.BlockSpec((_BLK, 1), lambda i: (i, 0)),
        ],
        out_specs=[
            pl.BlockSpec((G, 64), zero),
            pl.BlockSpec((G, 1), zero),
            pl.BlockSpec((G, 1), zero),
        ],
        out_shape=[
            jax.ShapeDtypeStruct((G, 64), jnp.float32),
            jax.ShapeDtypeStruct((G, 1), jnp.float32),
            jax.ShapeDtypeStruct((G, 1), jnp.float32),
        ],
    )(a3, t3, dinv, b3, Wl, bl, batch2d, ones_col)


def kernel(x, edge_index, batch, W1, b1, W2, b2, W3, b3, Wl, bl):
    ei = edge_index.astype(jnp.int32)
    src2d = ei[0].reshape(NCHUNKS, CHUNK)
    dst2d = ei[1].reshape(NCHUNKS, CHUNK)
    batch2d = batch.astype(jnp.int32).reshape(N, 1)
    xpad = jnp.pad(x, ((0, 0), (0, 16 - x.shape[1])))
    W1p = jnp.pad(W1, ((0, 16 - W1.shape[0]), (0, 0)))
    ones_rows = jnp.ones((CHUNK, 16), jnp.float32)
    ones_col = jnp.ones((N, 1), jnp.float32)
    z16 = jnp.zeros((RPT, 16), jnp.float32)
    z64 = jnp.zeros((RPT, 64), jnp.float32)
    z128 = jnp.zeros((RPT, 128), jnp.float32)

    g1 = _g1_call(xpad, W1p)
    degp = _deg_call(dst2d, ones_rows, z16)
    dinv, t1 = _prep_call(degp, g1)
    a1 = _agg_call(t1, src2d, dst2d, z128, 128)
    t2 = _mid1_call(a1, t1, dinv, b1.reshape(1, -1), W2)
    a2 = _agg_call(t2, src2d, dst2d, z128, 128)
    t3 = _mid2_call(a2, t2, dinv, b2.reshape(1, -1), W3)
    a3 = _agg_call(t3, src2d, dst2d, z64, 64)
    _, _, out = _pool_call(a3, t3, dinv, b3.reshape(1, -1),
                           Wl, bl.reshape(1, 1), batch2d, ones_col)
    return out


# speedup vs baseline: 1.0151x; 1.0151x over previous
import functools

import jax
import jax.numpy as jnp
from jax import lax
from jax.experimental import pallas as pl
from jax.experimental.pallas import tpu as pltpu
from jax.experimental.pallas import tpu_sc as plsc

N = 10000
E = 160000
G = 64
NC = 2
NS = 16
NW = NC * NS
CHUNK = 125
NCHUNKS = E // CHUNK
KPW = NCHUNKS // NW
NPAD = 10240
RPT = NPAD // NS

_HIGH = jax.lax.Precision.HIGHEST


def _sc_mesh():
    return plsc.VectorSubcoreMesh(core_axis_name="c", subcore_axis_name="s")


def _deg_call(dst2d, ones_rows, zrows):
    def body(dst_hbm, ones_hbm, z_hbm, out_hbm, didx, ones_v, sem, acc):
        c = lax.axis_index("c")
        s = lax.axis_index("s")
        w = s * NC + c
        pltpu.sync_copy(z_hbm, acc.at[pl.ds(s * RPT, RPT)])
        pltpu.sync_copy(dst_hbm.at[pl.ds(w * KPW, KPW)], didx)
        pltpu.sync_copy(ones_hbm, ones_v)
        plsc.subcore_barrier()

        def fire(k, carry):
            pltpu.async_copy(ones_v, acc.at[didx.at[k]], sem, add=True)
            return carry

        def drain(k, carry):
            pltpu.make_async_copy(ones_v, acc.at[didx.at[k]], sem).wait()
            return carry

        lax.fori_loop(0, KPW, fire, 0)
        lax.fori_loop(0, KPW, drain, 0)
        plsc.subcore_barrier()
        pltpu.sync_copy(acc.at[pl.ds(s * RPT, RPT)],
                        out_hbm.at[c, pl.ds(s * RPT, RPT)])

    call = pl.kernel(
        body,
        out_type=jax.ShapeDtypeStruct((NC, NPAD, 16), jnp.float32),
        mesh=_sc_mesh(),
        compiler_params=pltpu.CompilerParams(use_tc_tiling_on_sc=False),
        scratch_types=[
            pltpu.VMEM((KPW, CHUNK), jnp.int32),
            pltpu.VMEM((CHUNK, 16), jnp.float32),
            pltpu.SemaphoreType.DMA,
            pltpu.VMEM_SHARED((NPAD, 16), jnp.float32),
        ],
    )
    return call(dst2d, ones_rows, zrows)


def _agg_call(t, src2d, dst2d, zrows, d):
    _RING = 2 if d == 128 else 4

    def body(t_hbm, src_hbm, dst_hbm, z_hbm, out_hbm, *rest):
        sidx, didx = rest[0], rest[1]
        bufs = rest[2:2 + _RING]
        gs, ss, acc = rest[2 + _RING], rest[3 + _RING], rest[4 + _RING]
        c = lax.axis_index("c")
        s = lax.axis_index("s")
        w = s * NC + c
        pltpu.sync_copy(src_hbm.at[pl.ds(w * KPW, KPW)], sidx)
        pltpu.sync_copy(dst_hbm.at[pl.ds(w * KPW, KPW)], didx)
        for p in range(_RING):
            pltpu.async_copy(t_hbm.at[sidx.at[p]], bufs[p], gs.at[p])
        pltpu.sync_copy(z_hbm, acc.at[pl.ds(s * RPT, RPT)])
        plsc.subcore_barrier()

        def block(j, carry):
            for p in range(_RING):
                k = _RING * j + p
                pltpu.make_async_copy(t_hbm.at[sidx.at[k]], bufs[p],
                                      gs.at[p]).wait()
                pltpu.async_copy(bufs[p], acc.at[didx.at[k]], ss.at[p],
                                 add=True)

                @pl.when(k + _RING < KPW)
                def _():
                    pltpu.make_async_copy(bufs[p], acc.at[didx.at[k]],
                                          ss.at[p]).wait()
                    pltpu.async_copy(t_hbm.at[sidx.at[k + _RING]], bufs[p],
                                     gs.at[p])
            return carry

        lax.fori_loop(0, KPW // _RING, block, 0)
        for p in range(_RING):
            k = KPW - _RING + p
            pltpu.make_async_copy(bufs[p], acc.at[didx.at[k]], ss.at[p]).wait()
        plsc.subcore_barrier()
        pltpu.sync_copy(acc.at[pl.ds(s * RPT, RPT)],
                        out_hbm.at[c, pl.ds(s * RPT, RPT)])

    call = pl.kernel(
        body,
        out_type=jax.ShapeDtypeStruct((NC, NPAD, d), jnp.float32),
        mesh=_sc_mesh(),
        compiler_params=pltpu.CompilerParams(use_tc_tiling_on_sc=False),
        scratch_types=(
            [pltpu.VMEM((KPW, CHUNK), jnp.int32)] * 2
            + [pltpu.VMEM((CHUNK, d), jnp.float32)] * _RING
            + [pltpu.SemaphoreType.DMA((_RING,)),
               pltpu.SemaphoreType.DMA((_RING,)),
               pltpu.VMEM_SHARED((NPAD, d), jnp.float32)]
        ),
    )
    return call(t, src2d, dst2d, zrows)


_BLK = 2000
_GRID = N // _BLK


def _bf16_dot(a, b):
    return jnp.dot(a.astype(jnp.bfloat16), b.astype(jnp.bfloat16),
                   preferred_element_type=jnp.float32)


def _g1_body(x_ref, w1_ref, g1_ref):
    g1_ref[...] = _bf16_dot(x_ref[...], w1_ref[...])


def _g1_call(xpad, W1p):
    return pl.pallas_call(
        _g1_body,
        grid=(_GRID,),
        in_specs=[
            pl.BlockSpec((_BLK, 16), lambda i: (i, 0)),
            pl.BlockSpec((16, 128), lambda i: (0, 0)),
        ],
        out_specs=pl.BlockSpec((_BLK, 128), lambda i: (i, 0)),
        out_shape=jax.ShapeDtypeStruct((N, 128), jnp.float32),
    )(xpad, W1p)


def _prep_body(degp_ref, g1_ref, dinv_ref, t1_ref):
    deg = degp_ref[0, :, 0:1] + degp_ref[1, :, 0:1] + 1.0
    dinv = lax.rsqrt(deg)
    dinv_ref[...] = dinv
    t1_ref[...] = dinv * g1_ref[...]


def _prep_call(degp, g1):
    return pl.pallas_call(
        _prep_body,
        grid=(_GRID,),
        in_specs=[
            pl.BlockSpec((NC, _BLK, 16), lambda i: (0, i, 0)),
            pl.BlockSpec((_BLK, 128), lambda i: (i, 0)),
        ],
        out_specs=[
            pl.BlockSpec((_BLK, 1), lambda i: (i, 0)),
            pl.BlockSpec((_BLK, 128), lambda i: (i, 0)),
        ],
        out_shape=[
            jax.ShapeDtypeStruct((N, 1), jnp.float32),
            jax.ShapeDtypeStruct((N, 128), jnp.float32),
        ],
    )(degp, g1)


def _mid1_body(a1_ref, t1_ref, dinv_ref, b1_ref, w2_ref, t2_ref):
    agg = a1_ref[0] + a1_ref[1] + t1_ref[...]
    dinv = dinv_ref[...]
    h1 = jnp.maximum(dinv * agg + b1_ref[...], 0.0)
    t2_ref[...] = dinv * _bf16_dot(h1, w2_ref[...])


def _mid1_call(a1, t1, dinv, b1, W2):
    return pl.pallas_call(
        _mid1_body,
        grid=(_GRID,),
        in_specs=[
            pl.BlockSpec((NC, _BLK, 128), lambda i: (0, i, 0)),
            pl.BlockSpec((_BLK, 128), lambda i: (i, 0)),
            pl.BlockSpec((_BLK, 1), lambda i: (i, 0)),
            pl.BlockSpec((1, 128), lambda i: (0, 0)),
            pl.BlockSpec((128, 128), lambda i: (0, 0)),
        ],
        out_specs=pl.BlockSpec((_BLK, 128), lambda i: (i, 0)),
        out_shape=jax.ShapeDtypeStruct((N, 128), jnp.float32),
    )(a1, t1, dinv, b1, W2)


def _mid2_body(a2_ref, t2_ref, dinv_ref, b2_ref, w3_ref, t3_ref):
    agg = a2_ref[0] + a2_ref[1] + t2_ref[...]
    dinv = dinv_ref[...]
    h2 = jnp.maximum(dinv * agg + b2_ref[...], 0.0)
    t3_ref[...] = dinv * _bf16_dot(h2, w3_ref[...])


def _mid2_call(a2, t2, dinv, b2, W3):
    return pl.pallas_call(
        _mid2_body,
        grid=(_GRID,),
        in_specs=[
            pl.BlockSpec((NC, _BLK, 128), lambda i: (0, i, 0)),
            pl.BlockSpec((_BLK, 128), lambda i: (i, 0)),
            pl.BlockSpec((_BLK, 1), lambda i: (i, 0)),
            pl.BlockSpec((1, 128), lambda i: (0, 0)),
            pl.BlockSpec((128, 64), lambda i: (0, 0)),
        ],
        out_specs=pl.BlockSpec((_BLK, 64), lambda i: (i, 0)),
        out_shape=jax.ShapeDtypeStruct((N, 64), jnp.float32),
    )(a2, t2, dinv, b2, W3)


def _pool_body(a3_ref, t3_ref, dinv_ref, b3_ref, wl_ref, bl_ref, batch_ref,
               ones_ref, sums_ref, cnts_ref, out_ref):
    i = pl.program_id(0)
    agg = a3_ref[0] + a3_ref[1] + t3_ref[...]
    h3 = jnp.maximum(dinv_ref[...] * agg + b3_ref[...], 0.0)
    gid = lax.broadcasted_iota(jnp.int32, (_BLK, G), 1)
    onehot = (batch_ref[...] == gid).astype(jnp.float32)
    cdims = (((0,), (0,)), ((), ()))
    sm = lax.dot_general(onehot, h3, cdims, precision=_HIGH)
    ct = lax.dot_general(onehot, ones_ref[...], cdims, precision=_HIGH)

    @pl.when(i == 0)
    def _():
        sums_ref[...] = sm
        cnts_ref[...] = ct

    @pl.when(i > 0)
    def _():
        sums_ref[...] += sm
        cnts_ref[...] += ct

    @pl.when(i == pl.num_programs(0) - 1)
    def _():
        pooled = sums_ref[...] / jnp.maximum(cnts_ref[...], 1.0)
        out_ref[...] = _bf16_dot(pooled, wl_ref[...]) + bl_ref[...]


def _pool_call(a3, t3, dinv, b3, Wl, bl, batch2d, ones_col):
    zero = lambda i: (0, 0)
    return pl.pallas_call(
        _pool_body,
        grid=(_GRID,),
        in_specs=[
            pl.BlockSpec((NC, _BLK, 64), lambda i: (0, i, 0)),
            pl.BlockSpec((_BLK, 64), lambda i: (i, 0)),
            pl.BlockSpec((_BLK, 1), lambda i: (i, 0)),
            pl.BlockSpec((1, 64), zero),
            pl.BlockSpec((64, 1), zero),
            pl.BlockSpec((1, 1), zero),
            pl.BlockSpec((_BLK, 1), lambda i: (i, 0)),
            pl.BlockSpec((_BLK, 1), lambda i: (i, 0)),
        ],
        out_specs=[
            pl.BlockSpec((G, 64), zero),
            pl.BlockSpec((G, 1), zero),
            pl.BlockSpec((G, 1), zero),
        ],
        out_shape=[
            jax.ShapeDtypeStruct((G, 64), jnp.float32),
            jax.ShapeDtypeStruct((G, 1), jnp.float32),
            jax.ShapeDtypeStruct((G, 1), jnp.float32),
        ],
    )(a3, t3, dinv, b3, Wl, bl, batch2d, ones_col)


def kernel(x, edge_index, batch, W1, b1, W2, b2, W3, b3, Wl, bl):
    ei = edge_index.astype(jnp.int32)
    src2d = ei[0].reshape(NCHUNKS, CHUNK)
    dst2d = ei[1].reshape(NCHUNKS, CHUNK)
    batch2d = batch.astype(jnp.int32).reshape(N, 1)
    xpad = jnp.pad(x, ((0, 0), (0, 16 - x.shape[1])))
    W1p = jnp.pad(W1, ((0, 16 - W1.shape[0]), (0, 0)))
    ones_rows = jnp.ones((CHUNK, 16), jnp.float32)
    ones_col = jnp.ones((N, 1), jnp.float32)
    z16 = jnp.zeros((RPT, 16), jnp.float32)
    z64 = jnp.zeros((RPT, 64), jnp.float32)
    z128 = jnp.zeros((RPT, 128), jnp.float32)

    g1 = _g1_call(xpad, W1p)
    degp = _deg_call(dst2d, ones_rows, z16)
    dinv, t1 = _prep_call(degp, g1)
    a1 = _agg_call(t1, src2d, dst2d, z128, 128)
    t2 = _mid1_call(a1, t1, dinv, b1.reshape(1, -1), W2)
    a2 = _agg_call(t2, src2d, dst2d, z128, 128)
    t3 = _mid2_call(a2, t2, dinv, b2.reshape(1, -1), W3)
    a3 = _agg_call(t3, src2d, dst2d, z64, 64)
    _, _, out = _pool_call(a3, t3, dinv, b3.reshape(1, -1),
                           Wl, bl.reshape(1, 1), batch2d, ones_col)
    return out


# speedup vs baseline: 1.0327x; 1.0174x over previous
import functools

import jax
import jax.numpy as jnp
from jax import lax
from jax.experimental import pallas as pl
from jax.experimental.pallas import tpu as pltpu
from jax.experimental.pallas import tpu_sc as plsc

N = 10000
E = 160000
G = 64
NC = 2
NS = 16
NW = NC * NS
CHUNK = 125
NCHUNKS = E // CHUNK
KPW = NCHUNKS // NW
NPAD = 10240
RPT = NPAD // NS

_HIGH = jax.lax.Precision.HIGHEST


def _sc_mesh():
    return plsc.VectorSubcoreMesh(core_axis_name="c", subcore_axis_name="s")


def _deg_call(dst2d, ones_rows, zrows):
    def body(dst_hbm, ones_hbm, z_hbm, out_hbm, didx, ones_v, sem, acc):
        c = lax.axis_index("c")
        s = lax.axis_index("s")
        w = s * NC + c
        pltpu.sync_copy(z_hbm, acc.at[pl.ds(s * RPT, RPT)])
        pltpu.sync_copy(dst_hbm.at[pl.ds(w * KPW, KPW)], didx)
        pltpu.sync_copy(ones_hbm, ones_v)
        plsc.subcore_barrier()

        def fire(k, carry):
            pltpu.async_copy(ones_v, acc.at[didx.at[k]], sem, add=True)
            return carry

        def drain(k, carry):
            pltpu.make_async_copy(ones_v, acc.at[didx.at[k]], sem).wait()
            return carry

        lax.fori_loop(0, KPW, fire, 0)
        lax.fori_loop(0, KPW, drain, 0)
        plsc.subcore_barrier()
        pltpu.sync_copy(acc.at[pl.ds(s * RPT, RPT)],
                        out_hbm.at[c, pl.ds(s * RPT, RPT)])

    call = pl.kernel(
        body,
        out_type=jax.ShapeDtypeStruct((NC, NPAD, 16), jnp.float32),
        mesh=_sc_mesh(),
        compiler_params=pltpu.CompilerParams(use_tc_tiling_on_sc=False),
        scratch_types=[
            pltpu.VMEM((KPW, CHUNK), jnp.int32),
            pltpu.VMEM((CHUNK, 16), jnp.float32),
            pltpu.SemaphoreType.DMA,
            pltpu.VMEM_SHARED((NPAD, 16), jnp.float32),
        ],
    )
    return call(dst2d, ones_rows, zrows)


def _agg_call(t, src2d, dst2d, zrows, d):
    _RING = 2 if d == 128 else 4

    def body(t_hbm, src_hbm, dst_hbm, z_hbm, out_hbm, *rest):
        sidx, didx = rest[0], rest[1]
        bufs = rest[2:2 + _RING]
        gs, ss, acc = rest[2 + _RING], rest[3 + _RING], rest[4 + _RING]
        c = lax.axis_index("c")
        s = lax.axis_index("s")
        w = s * NC + c
        pltpu.sync_copy(src_hbm.at[pl.ds(w * KPW, KPW)], sidx)
        pltpu.sync_copy(dst_hbm.at[pl.ds(w * KPW, KPW)], didx)
        for p in range(_RING):
            pltpu.async_copy(t_hbm.at[sidx.at[p]], bufs[p], gs.at[p])
        pltpu.sync_copy(z_hbm, acc.at[pl.ds(s * RPT, RPT)])
        plsc.subcore_barrier()

        def block(j, carry):
            for p in range(_RING):
                k = _RING * j + p
                pltpu.make_async_copy(t_hbm.at[sidx.at[k]], bufs[p],
                                      gs.at[p]).wait()
                pltpu.async_copy(bufs[p], acc.at[didx.at[k]], ss.at[p],
                                 add=True)

                @pl.when(k + _RING < KPW)
                def _():
                    pltpu.make_async_copy(bufs[p], acc.at[didx.at[k]],
                                          ss.at[p]).wait()
                    pltpu.async_copy(t_hbm.at[sidx.at[k + _RING]], bufs[p],
                                     gs.at[p])
            return carry

        lax.fori_loop(0, KPW // _RING, block, 0)
        for p in range(_RING):
            k = KPW - _RING + p
            pltpu.make_async_copy(bufs[p], acc.at[didx.at[k]], ss.at[p]).wait()
        plsc.subcore_barrier()
        pltpu.sync_copy(acc.at[pl.ds(s * RPT, RPT)],
                        out_hbm.at[c, pl.ds(s * RPT, RPT)])

    call = pl.kernel(
        body,
        out_type=jax.ShapeDtypeStruct((NC, NPAD, d), jnp.float32),
        mesh=_sc_mesh(),
        compiler_params=pltpu.CompilerParams(use_tc_tiling_on_sc=False),
        scratch_types=(
            [pltpu.VMEM((KPW, CHUNK), jnp.int32)] * 2
            + [pltpu.VMEM((CHUNK, d), jnp.float32)] * _RING
            + [pltpu.SemaphoreType.DMA((_RING,)),
               pltpu.SemaphoreType.DMA((_RING,)),
               pltpu.VMEM_SHARED((NPAD, d), jnp.float32)]
        ),
    )
    return call(t, src2d, dst2d, zrows)


_BLK = 2000
_GRID = N // _BLK


def _bf16_dot(a, b):
    return jnp.dot(a.astype(jnp.bfloat16), b.astype(jnp.bfloat16),
                   preferred_element_type=jnp.float32)


def _prep_body(degp_ref, x_ref, w1_ref, dinv_ref, t1_ref):
    deg = degp_ref[0, :, 0:1] + degp_ref[1, :, 0:1] + 1.0
    dinv = lax.rsqrt(deg)
    dinv_ref[...] = dinv
    t1_ref[...] = dinv * _bf16_dot(x_ref[...], w1_ref[...])


def _prep_call(degp, xpad, W1p):
    return pl.pallas_call(
        _prep_body,
        grid=(_GRID,),
        in_specs=[
            pl.BlockSpec((NC, _BLK, 16), lambda i: (0, i, 0)),
            pl.BlockSpec((_BLK, 16), lambda i: (i, 0)),
            pl.BlockSpec((16, 128), lambda i: (0, 0)),
        ],
        out_specs=[
            pl.BlockSpec((_BLK, 1), lambda i: (i, 0)),
            pl.BlockSpec((_BLK, 128), lambda i: (i, 0)),
        ],
        out_shape=[
            jax.ShapeDtypeStruct((N, 1), jnp.float32),
            jax.ShapeDtypeStruct((N, 128), jnp.float32),
        ],
    )(degp, xpad, W1p)


def _mid1_body(a1_ref, t1_ref, dinv_ref, b1_ref, w2_ref, t2_ref):
    agg = a1_ref[0] + a1_ref[1] + t1_ref[...]
    dinv = dinv_ref[...]
    h1 = jnp.maximum(dinv * agg + b1_ref[...], 0.0)
    t2_ref[...] = dinv * _bf16_dot(h1, w2_ref[...])


def _mid1_call(a1, t1, dinv, b1, W2):
    return pl.pallas_call(
        _mid1_body,
        grid=(_GRID,),
        in_specs=[
            pl.BlockSpec((NC, _BLK, 128), lambda i: (0, i, 0)),
            pl.BlockSpec((_BLK, 128), lambda i: (i, 0)),
            pl.BlockSpec((_BLK, 1), lambda i: (i, 0)),
            pl.BlockSpec((1, 128), lambda i: (0, 0)),
            pl.BlockSpec((128, 128), lambda i: (0, 0)),
        ],
        out_specs=pl.BlockSpec((_BLK, 128), lambda i: (i, 0)),
        out_shape=jax.ShapeDtypeStruct((N, 128), jnp.float32),
    )(a1, t1, dinv, b1, W2)


def _mid2_body(a2_ref, t2_ref, dinv_ref, b2_ref, w3_ref, t3_ref):
    agg = a2_ref[0] + a2_ref[1] + t2_ref[...]
    dinv = dinv_ref[...]
    h2 = jnp.maximum(dinv * agg + b2_ref[...], 0.0)
    t3_ref[...] = dinv * _bf16_dot(h2, w3_ref[...])


def _mid2_call(a2, t2, dinv, b2, W3):
    return pl.pallas_call(
        _mid2_body,
        grid=(_GRID,),
        in_specs=[
            pl.BlockSpec((NC, _BLK, 128), lambda i: (0, i, 0)),
            pl.BlockSpec((_BLK, 128), lambda i: (i, 0)),
            pl.BlockSpec((_BLK, 1), lambda i: (i, 0)),
            pl.BlockSpec((1, 128), lambda i: (0, 0)),
            pl.BlockSpec((128, 64), lambda i: (0, 0)),
        ],
        out_specs=pl.BlockSpec((_BLK, 64), lambda i: (i, 0)),
        out_shape=jax.ShapeDtypeStruct((N, 64), jnp.float32),
    )(a2, t2, dinv, b2, W3)


def _pool_body(a3_ref, t3_ref, dinv_ref, b3_ref, wl_ref, bl_ref, batch_ref,
               ones_ref, sums_ref, cnts_ref, out_ref):
    i = pl.program_id(0)
    agg = a3_ref[0] + a3_ref[1] + t3_ref[...]
    h3 = jnp.maximum(dinv_ref[...] * agg + b3_ref[...], 0.0)
    gid = lax.broadcasted_iota(jnp.int32, (_BLK, G), 1)
    onehot = (batch_ref[...] == gid).astype(jnp.float32)
    cdims = (((0,), (0,)), ((), ()))
    sm = lax.dot_general(onehot, h3, cdims, precision=_HIGH)
    ct = lax.dot_general(onehot, ones_ref[...], cdims, precision=_HIGH)

    @pl.when(i == 0)
    def _():
        sums_ref[...] = sm
        cnts_ref[...] = ct

    @pl.when(i > 0)
    def _():
        sums_ref[...] += sm
        cnts_ref[...] += ct

    @pl.when(i == pl.num_programs(0) - 1)
    def _():
        pooled = sums_ref[...] / jnp.maximum(cnts_ref[...], 1.0)
        out_ref[...] = _bf16_dot(pooled, wl_ref[...]) + bl_ref[...]


def _pool_call(a3, t3, dinv, b3, Wl, bl, batch2d, ones_col):
    zero = lambda i: (0, 0)
    return pl.pallas_call(
        _pool_body,
        grid=(_GRID,),
        in_specs=[
            pl.BlockSpec((NC, _BLK, 64), lambda i: (0, i, 0)),
            pl.BlockSpec((_BLK, 64), lambda i: (i, 0)),
            pl.BlockSpec((_BLK, 1), lambda i: (i, 0)),
            pl.BlockSpec((1, 64), zero),
            pl.BlockSpec((64, 1), zero),
            pl.BlockSpec((1, 1), zero),
            pl.BlockSpec((_BLK, 1), lambda i: (i, 0)),
            pl.BlockSpec((_BLK, 1), lambda i: (i, 0)),
        ],
        out_specs=[
            pl.BlockSpec((G, 64), zero),
            pl.BlockSpec((G, 1), zero),
            pl.BlockSpec((G, 1), zero),
        ],
        out_shape=[
            jax.ShapeDtypeStruct((G, 64), jnp.float32),
            jax.ShapeDtypeStruct((G, 1), jnp.float32),
            jax.ShapeDtypeStruct((G, 1), jnp.float32),
        ],
    )(a3, t3, dinv, b3, Wl, bl, batch2d, ones_col)


def kernel(x, edge_index, batch, W1, b1, W2, b2, W3, b3, Wl, bl):
    ei = edge_index.astype(jnp.int32)
    src2d = ei[0].reshape(NCHUNKS, CHUNK)
    dst2d = ei[1].reshape(NCHUNKS, CHUNK)
    batch2d = batch.astype(jnp.int32).reshape(N, 1)
    xpad = jnp.pad(x, ((0, 0), (0, 16 - x.shape[1])))
    W1p = jnp.pad(W1, ((0, 16 - W1.shape[0]), (0, 0)))
    ones_rows = jnp.ones((CHUNK, 16), jnp.float32)
    ones_col = jnp.ones((N, 1), jnp.float32)
    z16 = jnp.zeros((RPT, 16), jnp.float32)
    z64 = jnp.zeros((RPT, 64), jnp.float32)
    z128 = jnp.zeros((RPT, 128), jnp.float32)

    degp = _deg_call(dst2d, ones_rows, z16)
    dinv, t1 = _prep_call(degp, xpad, W1p)
    a1 = _agg_call(t1, src2d, dst2d, z128, 128)
    t2 = _mid1_call(a1, t1, dinv, b1.reshape(1, -1), W2)
    a2 = _agg_call(t2, src2d, dst2d, z128, 128)
    t3 = _mid2_call(a2, t2, dinv, b2.reshape(1, -1), W3)
    a3 = _agg_call(t3, src2d, dst2d, z64, 64)
    _, _, out = _pool_call(a3, t3, dinv, b3.reshape(1, -1),
                           Wl, bl.reshape(1, 1), batch2d, ones_col)
    return out


# speedup vs baseline: 1.0643x; 1.0306x over previous
import functools

import jax
import jax.numpy as jnp
from jax import lax
from jax.experimental import pallas as pl
from jax.experimental.pallas import tpu as pltpu
from jax.experimental.pallas import tpu_sc as plsc

N = 10000
E = 160000
G = 64
NC = 2
NS = 16
NW = NC * NS
CHUNK = 125
NCHUNKS = E // CHUNK
KPW = NCHUNKS // NW
NPAD = 10240
RPT = NPAD // NS

_HIGH = jax.lax.Precision.HIGHEST


def _sc_mesh():
    return plsc.VectorSubcoreMesh(core_axis_name="c", subcore_axis_name="s")


def _deg_call(dst2d, ones_rows, zrows):
    def body(dst_hbm, ones_hbm, z_hbm, out_hbm, didx, ones_v, sem, acc):
        c = lax.axis_index("c")
        s = lax.axis_index("s")
        w = s * NC + c
        pltpu.sync_copy(z_hbm, acc.at[pl.ds(s * RPT, RPT)])
        pltpu.sync_copy(dst_hbm.at[pl.ds(w * KPW, KPW)], didx)
        pltpu.sync_copy(ones_hbm, ones_v)
        plsc.subcore_barrier()

        def fire(k, carry):
            pltpu.async_copy(ones_v, acc.at[didx.at[k]], sem, add=True)
            return carry

        def drain(k, carry):
            pltpu.make_async_copy(ones_v, acc.at[didx.at[k]], sem).wait()
            return carry

        lax.fori_loop(0, KPW, fire, 0)
        lax.fori_loop(0, KPW, drain, 0)
        plsc.subcore_barrier()
        pltpu.sync_copy(acc.at[pl.ds(s * RPT, RPT)],
                        out_hbm.at[c, pl.ds(s * RPT, RPT)])

    call = pl.kernel(
        body,
        out_type=jax.ShapeDtypeStruct((NC, NPAD, 16), jnp.float32),
        mesh=_sc_mesh(),
        compiler_params=pltpu.CompilerParams(use_tc_tiling_on_sc=False),
        scratch_types=[
            pltpu.VMEM((KPW, CHUNK), jnp.int32),
            pltpu.VMEM((CHUNK, 16), jnp.float32),
            pltpu.SemaphoreType.DMA,
            pltpu.VMEM_SHARED((NPAD, 16), jnp.float32),
        ],
    )
    return call(dst2d, ones_rows, zrows)


def _agg_call(t, src2d, dst2d, zrows, d, chunk=CHUNK, ring=4):
    _RING = ring
    nchunks = E // chunk
    kpw = nchunks // NW

    def body(t_hbm, src_hbm, dst_hbm, z_hbm, out_hbm, *rest):
        sidx, didx = rest[0], rest[1]
        bufs = rest[2:2 + _RING]
        gs, ss, acc = rest[2 + _RING], rest[3 + _RING], rest[4 + _RING]
        c = lax.axis_index("c")
        s = lax.axis_index("s")
        w = s * NC + c
        pltpu.sync_copy(src_hbm.at[pl.ds(w * kpw, kpw)], sidx)
        pltpu.sync_copy(dst_hbm.at[pl.ds(w * kpw, kpw)], didx)
        for p in range(_RING):
            pltpu.async_copy(t_hbm.at[sidx.at[p]], bufs[p], gs.at[p])
        pltpu.sync_copy(z_hbm, acc.at[pl.ds(s * RPT, RPT)])
        plsc.subcore_barrier()

        def block(j, carry):
            for p in range(_RING):
                k = _RING * j + p
                pltpu.make_async_copy(t_hbm.at[sidx.at[k]], bufs[p],
                                      gs.at[p]).wait()
                pltpu.async_copy(bufs[p], acc.at[didx.at[k]], ss.at[p],
                                 add=True)

                @pl.when(k + _RING < kpw)
                def _():
                    pltpu.make_async_copy(bufs[p], acc.at[didx.at[k]],
                                          ss.at[p]).wait()
                    pltpu.async_copy(t_hbm.at[sidx.at[k + _RING]], bufs[p],
                                     gs.at[p])
            return carry

        lax.fori_loop(0, kpw // _RING, block, 0)
        for p in range(_RING):
            k = kpw - _RING + p
            pltpu.make_async_copy(bufs[p], acc.at[didx.at[k]], ss.at[p]).wait()
        plsc.subcore_barrier()
        pltpu.sync_copy(acc.at[pl.ds(s * RPT, RPT)],
                        out_hbm.at[c, pl.ds(s * RPT, RPT)])

    call = pl.kernel(
        body,
        out_type=jax.ShapeDtypeStruct((NC, NPAD, d), jnp.float32),
        mesh=_sc_mesh(),
        compiler_params=pltpu.CompilerParams(use_tc_tiling_on_sc=False),
        scratch_types=(
            [pltpu.VMEM((kpw, chunk), jnp.int32)] * 2
            + [pltpu.VMEM((chunk, d), jnp.float32)] * _RING
            + [pltpu.SemaphoreType.DMA((_RING,)),
               pltpu.SemaphoreType.DMA((_RING,)),
               pltpu.VMEM_SHARED((NPAD, d), jnp.float32)]
        ),
    )
    return call(t, src2d, dst2d, zrows)


_BLK = 2000
_GRID = N // _BLK


def _bf16_dot(a, b):
    return jnp.dot(a.astype(jnp.bfloat16), b.astype(jnp.bfloat16),
                   preferred_element_type=jnp.float32)


def _prep_body(degp_ref, x_ref, w1_ref, dinv_ref, t1_ref):
    deg = degp_ref[0, :, 0:1] + degp_ref[1, :, 0:1] + 1.0
    dinv = lax.rsqrt(deg)
    dinv_ref[...] = dinv
    t1_ref[...] = dinv * _bf16_dot(x_ref[...], w1_ref[...])


def _prep_call(degp, xpad, W1p):
    return pl.pallas_call(
        _prep_body,
        grid=(_GRID,),
        in_specs=[
            pl.BlockSpec((NC, _BLK, 16), lambda i: (0, i, 0)),
            pl.BlockSpec((_BLK, 16), lambda i: (i, 0)),
            pl.BlockSpec((16, 128), lambda i: (0, 0)),
        ],
        out_specs=[
            pl.BlockSpec((_BLK, 1), lambda i: (i, 0)),
            pl.BlockSpec((_BLK, 128), lambda i: (i, 0)),
        ],
        out_shape=[
            jax.ShapeDtypeStruct((N, 1), jnp.float32),
            jax.ShapeDtypeStruct((N, 128), jnp.float32),
        ],
    )(degp, xpad, W1p)


def _mid1_body(a1_ref, t1_ref, dinv_ref, b1_ref, w2_ref, t2_ref):
    agg = a1_ref[0] + a1_ref[1] + t1_ref[...]
    dinv = dinv_ref[...]
    h1 = jnp.maximum(dinv * agg + b1_ref[...], 0.0)
    t2_ref[...] = dinv * _bf16_dot(h1, w2_ref[...])


def _mid1_call(a1, t1, dinv, b1, W2):
    return pl.pallas_call(
        _mid1_body,
        grid=(_GRID,),
        in_specs=[
            pl.BlockSpec((NC, _BLK, 128), lambda i: (0, i, 0)),
            pl.BlockSpec((_BLK, 128), lambda i: (i, 0)),
            pl.BlockSpec((_BLK, 1), lambda i: (i, 0)),
            pl.BlockSpec((1, 128), lambda i: (0, 0)),
            pl.BlockSpec((128, 128), lambda i: (0, 0)),
        ],
        out_specs=pl.BlockSpec((_BLK, 128), lambda i: (i, 0)),
        out_shape=jax.ShapeDtypeStruct((N, 128), jnp.float32),
    )(a1, t1, dinv, b1, W2)


def _mid2_body(a2_ref, t2_ref, dinv_ref, b2_ref, w3_ref, t3_ref):
    agg = a2_ref[0] + a2_ref[1] + t2_ref[...]
    dinv = dinv_ref[...]
    h2 = jnp.maximum(dinv * agg + b2_ref[...], 0.0)
    t3_ref[...] = dinv * _bf16_dot(h2, w3_ref[...])


def _mid2_call(a2, t2, dinv, b2, W3):
    return pl.pallas_call(
        _mid2_body,
        grid=(_GRID,),
        in_specs=[
            pl.BlockSpec((NC, _BLK, 128), lambda i: (0, i, 0)),
            pl.BlockSpec((_BLK, 128), lambda i: (i, 0)),
            pl.BlockSpec((_BLK, 1), lambda i: (i, 0)),
            pl.BlockSpec((1, 128), lambda i: (0, 0)),
            pl.BlockSpec((128, 64), lambda i: (0, 0)),
        ],
        out_specs=pl.BlockSpec((_BLK, 64), lambda i: (i, 0)),
        out_shape=jax.ShapeDtypeStruct((N, 64), jnp.float32),
    )(a2, t2, dinv, b2, W3)


def _pool_body(a3_ref, t3_ref, dinv_ref, b3_ref, wl_ref, bl_ref, batch_ref,
               ones_ref, sums_ref, cnts_ref, out_ref):
    i = pl.program_id(0)
    agg = a3_ref[0] + a3_ref[1] + t3_ref[...]
    h3 = jnp.maximum(dinv_ref[...] * agg + b3_ref[...], 0.0)
    gid = lax.broadcasted_iota(jnp.int32, (_BLK, G), 1)
    onehot = (batch_ref[...] == gid).astype(jnp.float32)
    cdims = (((0,), (0,)), ((), ()))
    sm = lax.dot_general(onehot, h3, cdims, precision=_HIGH)
    ct = lax.dot_general(onehot, ones_ref[...], cdims, precision=_HIGH)

    @pl.when(i == 0)
    def _():
        sums_ref[...] = sm
        cnts_ref[...] = ct

    @pl.when(i > 0)
    def _():
        sums_ref[...] += sm
        cnts_ref[...] += ct

    @pl.when(i == pl.num_programs(0) - 1)
    def _():
        pooled = sums_ref[...] / jnp.maximum(cnts_ref[...], 1.0)
        out_ref[...] = _bf16_dot(pooled, wl_ref[...]) + bl_ref[...]


def _pool_call(a3, t3, dinv, b3, Wl, bl, batch2d, ones_col):
    zero = lambda i: (0, 0)
    return pl.pallas_call(
        _pool_body,
        grid=(_GRID,),
        in_specs=[
            pl.BlockSpec((NC, _BLK, 64), lambda i: (0, i, 0)),
            pl.BlockSpec((_BLK, 64), lambda i: (i, 0)),
            pl.BlockSpec((_BLK, 1), lambda i: (i, 0)),
            pl.BlockSpec((1, 64), zero),
            pl.BlockSpec((64, 1), zero),
            pl.BlockSpec((1, 1), zero),
            pl.BlockSpec((_BLK, 1), lambda i: (i, 0)),
            pl.BlockSpec((_BLK, 1), lambda i: (i, 0)),
        ],
        out_specs=[
            pl.BlockSpec((G, 64), zero),
            pl.BlockSpec((G, 1), zero),
            pl.BlockSpec((G, 1), zero),
        ],
        out_shape=[
            jax.ShapeDtypeStruct((G, 64), jnp.float32),
            jax.ShapeDtypeStruct((G, 1), jnp.float32),
            jax.ShapeDtypeStruct((G, 1), jnp.float32),
        ],
    )(a3, t3, dinv, b3, Wl, bl, batch2d, ones_col)


def kernel(x, edge_index, batch, W1, b1, W2, b2, W3, b3, Wl, bl):
    ei = edge_index.astype(jnp.int32)
    src2d = ei[0].reshape(NCHUNKS, CHUNK)
    dst2d = ei[1].reshape(NCHUNKS, CHUNK)
    batch2d = batch.astype(jnp.int32).reshape(N, 1)
    xpad = jnp.pad(x, ((0, 0), (0, 16 - x.shape[1])))
    W1p = jnp.pad(W1, ((0, 16 - W1.shape[0]), (0, 0)))
    ones_rows = jnp.ones((CHUNK, 16), jnp.float32)
    ones_col = jnp.ones((N, 1), jnp.float32)
    z16 = jnp.zeros((RPT, 16), jnp.float32)
    z64 = jnp.zeros((RPT, 64), jnp.float32)
    z128 = jnp.zeros((RPT, 128), jnp.float32)

    src50 = ei[0].reshape(E // 50, 50)
    dst50 = ei[1].reshape(E // 50, 50)
    degp = _deg_call(dst2d, ones_rows, z16)
    dinv, t1 = _prep_call(degp, xpad, W1p)
    a1 = _agg_call(t1, src50, dst50, z128, 128, chunk=50, ring=4)
    t2 = _mid1_call(a1, t1, dinv, b1.reshape(1, -1), W2)
    a2 = _agg_call(t2, src50, dst50, z128, 128, chunk=50, ring=4)
    t3 = _mid2_call(a2, t2, dinv, b2.reshape(1, -1), W3)
    a3 = _agg_call(t3, src2d, dst2d, z64, 64)
    _, _, out = _pool_call(a3, t3, dinv, b3.reshape(1, -1),
                           Wl, bl.reshape(1, 1), batch2d, ones_col)
    return out


# speedup vs baseline: 1.0709x; 1.0062x over previous
import functools

import jax
import jax.numpy as jnp
from jax import lax
from jax.experimental import pallas as pl
from jax.experimental.pallas import tpu as pltpu
from jax.experimental.pallas import tpu_sc as plsc

N = 10000
E = 160000
G = 64
NC = 2
NS = 16
NW = NC * NS
CHUNK = 125
NCHUNKS = E // CHUNK
KPW = NCHUNKS // NW
NPAD = 10240
RPT = NPAD // NS

_HIGH = jax.lax.Precision.HIGHEST


def _sc_mesh():
    return plsc.VectorSubcoreMesh(core_axis_name="c", subcore_axis_name="s")


def _deg_call(dst2d, ones_rows, zrows):
    def body(dst_hbm, ones_hbm, z_hbm, out_hbm, didx, ones_v, sem, acc):
        c = lax.axis_index("c")
        s = lax.axis_index("s")
        w = s * NC + c
        pltpu.sync_copy(z_hbm, acc.at[pl.ds(s * RPT, RPT)])
        pltpu.sync_copy(dst_hbm.at[pl.ds(w * KPW, KPW)], didx)
        pltpu.sync_copy(ones_hbm, ones_v)
        plsc.subcore_barrier()

        def fire(k, carry):
            pltpu.async_copy(ones_v, acc.at[didx.at[k]], sem, add=True)
            return carry

        def drain(k, carry):
            pltpu.make_async_copy(ones_v, acc.at[didx.at[k]], sem).wait()
            return carry

        lax.fori_loop(0, KPW, fire, 0)
        lax.fori_loop(0, KPW, drain, 0)
        plsc.subcore_barrier()
        pltpu.sync_copy(acc.at[pl.ds(s * RPT, RPT)],
                        out_hbm.at[c, pl.ds(s * RPT, RPT)])

    call = pl.kernel(
        body,
        out_type=jax.ShapeDtypeStruct((NC, NPAD, 16), jnp.float32),
        mesh=_sc_mesh(),
        compiler_params=pltpu.CompilerParams(use_tc_tiling_on_sc=False),
        scratch_types=[
            pltpu.VMEM((KPW, CHUNK), jnp.int32),
            pltpu.VMEM((CHUNK, 16), jnp.float32),
            pltpu.SemaphoreType.DMA,
            pltpu.VMEM_SHARED((NPAD, 16), jnp.float32),
        ],
    )
    return call(dst2d, ones_rows, zrows)


def _agg_call(t, src2d, dst2d, zrows, d, chunk=CHUNK, ring=4):
    _RING = ring
    nchunks = E // chunk
    kpw = nchunks // NW

    def body(t_hbm, src_hbm, dst_hbm, z_hbm, out_hbm, *rest):
        sidx, didx = rest[0], rest[1]
        bufs = rest[2:2 + _RING]
        gs, ss, acc = rest[2 + _RING], rest[3 + _RING], rest[4 + _RING]
        c = lax.axis_index("c")
        s = lax.axis_index("s")
        w = s * NC + c
        pltpu.sync_copy(src_hbm.at[pl.ds(w * kpw, kpw)], sidx)
        pltpu.sync_copy(dst_hbm.at[pl.ds(w * kpw, kpw)], didx)
        for p in range(_RING):
            pltpu.async_copy(t_hbm.at[sidx.at[p]], bufs[p], gs.at[p])
        pltpu.sync_copy(z_hbm, acc.at[pl.ds(s * RPT, RPT)])
        plsc.subcore_barrier()

        def block(j, carry):
            for p in range(_RING):
                k = _RING * j + p
                pltpu.make_async_copy(t_hbm.at[sidx.at[k]], bufs[p],
                                      gs.at[p]).wait()
                pltpu.async_copy(bufs[p], acc.at[didx.at[k]], ss.at[p],
                                 add=True)

                @pl.when(k + _RING < kpw)
                def _():
                    pltpu.make_async_copy(bufs[p], acc.at[didx.at[k]],
                                          ss.at[p]).wait()
                    pltpu.async_copy(t_hbm.at[sidx.at[k + _RING]], bufs[p],
                                     gs.at[p])
            return carry

        lax.fori_loop(0, kpw // _RING, block, 0)
        for p in range(_RING):
            k = kpw - _RING + p
            pltpu.make_async_copy(bufs[p], acc.at[didx.at[k]], ss.at[p]).wait()
        plsc.subcore_barrier()
        pltpu.sync_copy(acc.at[pl.ds(s * RPT, RPT)],
                        out_hbm.at[c, pl.ds(s * RPT, RPT)])

    call = pl.kernel(
        body,
        out_type=jax.ShapeDtypeStruct((NC, NPAD, d), jnp.float32),
        mesh=_sc_mesh(),
        compiler_params=pltpu.CompilerParams(use_tc_tiling_on_sc=False),
        scratch_types=(
            [pltpu.VMEM((kpw, chunk), jnp.int32)] * 2
            + [pltpu.VMEM((chunk, d), jnp.float32)] * _RING
            + [pltpu.SemaphoreType.DMA((_RING,)),
               pltpu.SemaphoreType.DMA((_RING,)),
               pltpu.VMEM_SHARED((NPAD, d), jnp.float32)]
        ),
    )
    return call(t, src2d, dst2d, zrows)


_BLK = 2000
_GRID = N // _BLK


def _bf16_dot(a, b):
    return jnp.dot(a.astype(jnp.bfloat16), b.astype(jnp.bfloat16),
                   preferred_element_type=jnp.float32)


def _prep_body(degp_ref, x_ref, w1_ref, dinv_ref, t1_ref):
    deg = degp_ref[0, :, 0:1] + degp_ref[1, :, 0:1] + 1.0
    dinv = lax.rsqrt(deg)
    dinv_ref[...] = dinv
    t1_ref[...] = dinv * _bf16_dot(x_ref[...], w1_ref[...])


def _prep_call(degp, xpad, W1p):
    return pl.pallas_call(
        _prep_body,
        grid=(_GRID,),
        in_specs=[
            pl.BlockSpec((NC, _BLK, 16), lambda i: (0, i, 0)),
            pl.BlockSpec((_BLK, 16), lambda i: (i, 0)),
            pl.BlockSpec((16, 128), lambda i: (0, 0)),
        ],
        out_specs=[
            pl.BlockSpec((_BLK, 1), lambda i: (i, 0)),
            pl.BlockSpec((_BLK, 128), lambda i: (i, 0)),
        ],
        out_shape=[
            jax.ShapeDtypeStruct((N, 1), jnp.float32),
            jax.ShapeDtypeStruct((N, 128), jnp.float32),
        ],
    )(degp, xpad, W1p)


def _mid1_body(a1_ref, t1_ref, dinv_ref, b1_ref, w2_ref, t2_ref):
    agg = a1_ref[0] + a1_ref[1] + t1_ref[...]
    dinv = dinv_ref[...]
    h1 = jnp.maximum(dinv * agg + b1_ref[...], 0.0)
    t2_ref[...] = dinv * _bf16_dot(h1, w2_ref[...])


def _mid1_call(a1, t1, dinv, b1, W2):
    return pl.pallas_call(
        _mid1_body,
        grid=(_GRID,),
        in_specs=[
            pl.BlockSpec((NC, _BLK, 128), lambda i: (0, i, 0)),
            pl.BlockSpec((_BLK, 128), lambda i: (i, 0)),
            pl.BlockSpec((_BLK, 1), lambda i: (i, 0)),
            pl.BlockSpec((1, 128), lambda i: (0, 0)),
            pl.BlockSpec((128, 128), lambda i: (0, 0)),
        ],
        out_specs=pl.BlockSpec((_BLK, 128), lambda i: (i, 0)),
        out_shape=jax.ShapeDtypeStruct((N, 128), jnp.float32),
    )(a1, t1, dinv, b1, W2)


def _mid2_body(a2_ref, t2_ref, dinv_ref, b2_ref, w3_ref, t3_ref):
    agg = a2_ref[0] + a2_ref[1] + t2_ref[...]
    dinv = dinv_ref[...]
    h2 = jnp.maximum(dinv * agg + b2_ref[...], 0.0)
    t3_ref[...] = dinv * _bf16_dot(h2, w3_ref[...])


def _mid2_call(a2, t2, dinv, b2, W3):
    return pl.pallas_call(
        _mid2_body,
        grid=(_GRID,),
        in_specs=[
            pl.BlockSpec((NC, _BLK, 128), lambda i: (0, i, 0)),
            pl.BlockSpec((_BLK, 128), lambda i: (i, 0)),
            pl.BlockSpec((_BLK, 1), lambda i: (i, 0)),
            pl.BlockSpec((1, 128), lambda i: (0, 0)),
            pl.BlockSpec((128, 64), lambda i: (0, 0)),
        ],
        out_specs=pl.BlockSpec((_BLK, 64), lambda i: (i, 0)),
        out_shape=jax.ShapeDtypeStruct((N, 64), jnp.float32),
    )(a2, t2, dinv, b2, W3)


def _pool_body(a3_ref, t3_ref, dinv_ref, b3_ref, wl_ref, bl_ref, batch_ref,
               ones_ref, sums_ref, cnts_ref, out_ref):
    i = pl.program_id(0)
    agg = a3_ref[0] + a3_ref[1] + t3_ref[...]
    h3 = jnp.maximum(dinv_ref[...] * agg + b3_ref[...], 0.0)
    gid = lax.broadcasted_iota(jnp.int32, (_BLK, G), 1)
    onehot = (batch_ref[...] == gid).astype(jnp.float32)
    cdims = (((0,), (0,)), ((), ()))
    sm = lax.dot_general(onehot, h3, cdims, precision=_HIGH)
    ct = lax.dot_general(onehot, ones_ref[...], cdims, precision=_HIGH)

    @pl.when(i == 0)
    def _():
        sums_ref[...] = sm
        cnts_ref[...] = ct

    @pl.when(i > 0)
    def _():
        sums_ref[...] += sm
        cnts_ref[...] += ct

    @pl.when(i == pl.num_programs(0) - 1)
    def _():
        pooled = sums_ref[...] / jnp.maximum(cnts_ref[...], 1.0)
        out_ref[...] = _bf16_dot(pooled, wl_ref[...]) + bl_ref[...]


def _pool_call(a3, t3, dinv, b3, Wl, bl, batch2d, ones_col):
    zero = lambda i: (0, 0)
    return pl.pallas_call(
        _pool_body,
        grid=(_GRID,),
        in_specs=[
            pl.BlockSpec((NC, _BLK, 64), lambda i: (0, i, 0)),
            pl.BlockSpec((_BLK, 64), lambda i: (i, 0)),
            pl.BlockSpec((_BLK, 1), lambda i: (i, 0)),
            pl.BlockSpec((1, 64), zero),
            pl.BlockSpec((64, 1), zero),
            pl.BlockSpec((1, 1), zero),
            pl.BlockSpec((_BLK, 1), lambda i: (i, 0)),
            pl.BlockSpec((_BLK, 1), lambda i: (i, 0)),
        ],
        out_specs=[
            pl.BlockSpec((G, 64), zero),
            pl.BlockSpec((G, 1), zero),
            pl.BlockSpec((G, 1), zero),
        ],
        out_shape=[
            jax.ShapeDtypeStruct((G, 64), jnp.float32),
            jax.ShapeDtypeStruct((G, 1), jnp.float32),
            jax.ShapeDtypeStruct((G, 1), jnp.float32),
        ],
    )(a3, t3, dinv, b3, Wl, bl, batch2d, ones_col)


def kernel(x, edge_index, batch, W1, b1, W2, b2, W3, b3, Wl, bl):
    ei = edge_index.astype(jnp.int32)
    src2d = ei[0].reshape(NCHUNKS, CHUNK)
    dst2d = ei[1].reshape(NCHUNKS, CHUNK)
    batch2d = batch.astype(jnp.int32).reshape(N, 1)
    xpad = jnp.pad(x, ((0, 0), (0, 16 - x.shape[1])))
    W1p = jnp.pad(W1, ((0, 16 - W1.shape[0]), (0, 0)))
    ones_rows = jnp.ones((CHUNK, 16), jnp.float32)
    ones_col = jnp.ones((N, 1), jnp.float32)
    z16 = jnp.zeros((RPT, 16), jnp.float32)
    z64 = jnp.zeros((RPT, 64), jnp.float32)
    z128 = jnp.zeros((RPT, 128), jnp.float32)

    src50 = ei[0].reshape(E // 50, 50)
    dst50 = ei[1].reshape(E // 50, 50)
    degp = _deg_call(dst2d, ones_rows, z16)
    dinv, t1 = _prep_call(degp, xpad, W1p)
    a1 = _agg_call(t1, src50, dst50, z128, 128, chunk=50, ring=5)
    t2 = _mid1_call(a1, t1, dinv, b1.reshape(1, -1), W2)
    a2 = _agg_call(t2, src50, dst50, z128, 128, chunk=50, ring=5)
    t3 = _mid2_call(a2, t2, dinv, b2.reshape(1, -1), W3)
    a3 = _agg_call(t3, src2d, dst2d, z64, 64, ring=5)
    _, _, out = _pool_call(a3, t3, dinv, b3.reshape(1, -1),
                           Wl, bl.reshape(1, 1), batch2d, ones_col)
    return out
